# baseline (speedup 1.0000x reference)
import jax
import jax.numpy as jnp
from jax.experimental import pallas as pl
from jax.experimental.pallas import tpu as pltpu

_E = 64
_K = 8
_ALPHA = 0.01


def _router_block(x_ref, wt_ref, tw_ref, ti_ref, ssum_ref, cnt_ref):
    x = x_ref[...]
    wt = wt_ref[...]
    logits = jnp.dot(x, wt, preferred_element_type=jnp.float32)

    m = jnp.max(logits, axis=1, keepdims=True)
    ex = jnp.exp(logits - m)
    z = jnp.sum(ex, axis=1, keepdims=True)
    scores = (ex / z).T

    iota_f = jax.lax.broadcasted_iota(jnp.int32, scores.shape, 0).astype(
        jnp.float32)
    sbits = jax.lax.bitcast_convert_type(scores, jnp.int32)
    work = sbits
    mks = []
    idxs = []
    for _ in range(_K):
        mk = jnp.max(work, axis=0, keepdims=True)
        eq = work == mk
        idxs.append(jnp.sum(jnp.where(eq, iota_f, 0.0), axis=0,
                            keepdims=True))
        work = jnp.where(eq, jnp.int32(-(2**31)), work)
        mks.append(mk)

    mkcat = jnp.concatenate(mks, axis=0)
    ti = jnp.concatenate(idxs, axis=0).astype(jnp.int32)
    tw = jax.lax.bitcast_convert_type(mkcat, jnp.float32)
    tw = tw / (jnp.sum(tw, axis=0, keepdims=True) + 1e-9)

    tw_ref[...] = tw
    ti_ref[...] = ti

    hits = (sbits >= mks[-1]).astype(jnp.float32)
    ssum_ref[...] = jnp.sum(scores, axis=1, keepdims=True).reshape(1, _E, 1)
    cnt_ref[...] = jnp.sum(hits, axis=1, keepdims=True).reshape(1, _E, 1)


def _aux_reduce(ssum_ref, cnt_ref, aux_ref, *, t_total):
    ssum = jnp.sum(ssum_ref[...], axis=0)
    cnt = jnp.sum(cnt_ref[...], axis=0)
    scale = _ALPHA * _E / (float(t_total) * float(t_total) * _K)
    aux_ref[...] = jnp.sum(ssum * cnt, axis=0, keepdims=True) * scale


def kernel(x, W):
    bsz, seq, d = x.shape
    t = bsz * seq
    xf = x.reshape(t, d)
    wt = W.T

    blk = 1024
    nblk = t // blk

    tw_kt, ti_kt, ssum_p, cnt_p = pl.pallas_call(
        _router_block,
        grid=(nblk,),
        in_specs=[
            pl.BlockSpec((blk, d), lambda i: (i, 0)),
            pl.BlockSpec((d, _E), lambda i: (0, 0)),
        ],
        out_specs=[
            pl.BlockSpec((_K, blk), lambda i: (0, i)),
            pl.BlockSpec((_K, blk), lambda i: (0, i)),
            pl.BlockSpec((1, _E, 1), lambda i: (i, 0, 0)),
            pl.BlockSpec((1, _E, 1), lambda i: (i, 0, 0)),
        ],
        out_shape=[
            jax.ShapeDtypeStruct((_K, t), jnp.float32),
            jax.ShapeDtypeStruct((_K, t), jnp.int32),
            jax.ShapeDtypeStruct((nblk, _E, 1), jnp.float32),
            jax.ShapeDtypeStruct((nblk, _E, 1), jnp.float32),
        ],
        compiler_params=pltpu.CompilerParams(
            dimension_semantics=("parallel",),
        ),
    )(xf, wt)

    import functools
    aux = pl.pallas_call(
        functools.partial(_aux_reduce, t_total=t),
        out_shape=jax.ShapeDtypeStruct((1, 1), jnp.float32),
    )(ssum_p, cnt_p)

    return tw_kt.T, ti_kt.T, aux[0, 0]
